# Initial kernel scaffold; baseline (speedup 1.0000x reference)
import jax
import jax.numpy as jnp
from jax.experimental import pallas as pl


def kernel(x):
    raise NotImplementedError("write your pallas kernel here")


# speedup vs baseline: 7.6335x; 7.6335x over previous
import functools

import jax
import jax.numpy as jnp
from jax import lax
from jax.experimental import pallas as pl
from jax.experimental.pallas import tpu as pltpu


def _dcp_body(k, h, w, x_ref, o_ref, bits_ref):
    imsz = h * w
    x0 = x_ref[0, 0]
    x1 = x_ref[0, 1]
    x2 = x_ref[0, 2]
    dark = jnp.minimum(jnp.minimum(x0, x1), x2)
    bits_ref[...] = lax.bitcast_convert_type(dark, jnp.int32)

    def bis1(_, carry):
        lo, hi = carry
        mid = (lo + hi) >> 1
        cnt = jnp.sum((bits_ref[...] >= mid).astype(jnp.int32))
        ge = cnt >= k
        return jnp.where(ge, mid, lo), jnp.where(ge, hi, mid)

    tbits, _ = lax.fori_loop(
        0, 30, bis1, (jnp.int32(0), jnp.int32(0x40000000))
    )

    n_strict = jnp.sum((bits_ref[...] > tbits).astype(jnp.int32))
    r = k - n_strict

    def lin_idx():
        rows = lax.broadcasted_iota(jnp.int32, (h, w), 0)
        cols = lax.broadcasted_iota(jnp.int32, (h, w), 1)
        return rows * w + cols

    def bis2(_, carry):
        lo2, hi2 = carry
        mid = (lo2 + hi2) >> 1
        tie = bits_ref[...] == tbits
        cnt = jnp.sum((tie & (lin_idx() < mid)).astype(jnp.int32))
        ge = cnt >= r
        return jnp.where(ge, lo2, mid), jnp.where(ge, mid, hi2)

    _, jstar = lax.fori_loop(0, 18, bis2, (jnp.int32(0), jnp.int32(imsz)))

    bits = bits_ref[...]
    sel = (bits > tbits) | ((bits == tbits) & (lin_idx() < jstar))

    s0 = jnp.sum(jnp.where(sel, x0, 0.0))
    s1 = jnp.sum(jnp.where(sel, x1, 0.0))
    s2 = jnp.sum(jnp.where(sel, x2, 0.0))
    inv_k = 1.0 / k
    a0 = s0 * inv_k + 1e-6
    a1 = s1 * inv_k + 1e-6
    a2 = s2 * inv_k + 1e-6

    m = jnp.minimum(
        jnp.minimum(x0 * (1.0 / a0), x1 * (1.0 / a1)), x2 * (1.0 / a2)
    )
    tx = 1.0 - 0.75 * m
    recip = 1.0 / jnp.maximum(tx, 0.1)
    o_ref[0, 0] = (x0 - a0) * recip + a0
    o_ref[0, 1] = (x1 - a1) * recip + a1
    o_ref[0, 2] = (x2 - a2) * recip + a2


def _dcp(x, interpret=False):
    b, c, h, w = x.shape
    imsz = h * w
    k = max(imsz // 1000, 1)
    return pl.pallas_call(
        functools.partial(_dcp_body, k, h, w),
        grid=(b,),
        in_specs=[pl.BlockSpec((1, c, h, w), lambda i: (i, 0, 0, 0))],
        out_specs=pl.BlockSpec((1, c, h, w), lambda i: (i, 0, 0, 0)),
        out_shape=jax.ShapeDtypeStruct(x.shape, x.dtype),
        scratch_shapes=[pltpu.VMEM((h, w), jnp.int32)],
        interpret=interpret,
    )(x)


def kernel(x):
    return _dcp(x)
